# Initial kernel scaffold; baseline (speedup 1.0000x reference)
import jax
import jax.numpy as jnp
from jax.experimental import pallas as pl


def kernel(x, edge_index, edge_list, target_indices, emb, conv_W, conv_b, conv_g, conv_beta, lin_W, lin_b, lin_g, lin_beta, out_W, out_b):
    raise NotImplementedError("write your pallas kernel here")


# speedup vs baseline: 4.4885x; 4.4885x over previous
import functools

import jax
import jax.numpy as jnp
from jax import lax
from jax.experimental import pallas as pl
from jax.experimental.pallas import tpu as pltpu
from jax.experimental.pallas import tpu_sc as plsc

N = 10000
NP = 10240
D = 128
E = 320000
T = 1024
EPS = 1e-5

NC = 2
NS = 16
NW = NC * NS
K = 128
NB2 = 158
EP = NS * NB2 * K
RPT = NP // NW
XROWS = 8

SPLIT = 5120
ACC_R = 5136
WR = SPLIT // NS
ZR = ACC_R // NS

BR = 512
GRID = NP // BR


def _mesh():
    return plsc.VectorSubcoreMesh(core_axis_name="c", subcore_axis_name="s")


def _sc_prep(x3, emb):
    @functools.partial(
        pl.kernel,
        out_type=jax.ShapeDtypeStruct((NP, D), jnp.float32),
        mesh=_mesh(),
        scratch_types=[
            pltpu.VMEM((XROWS, K), jnp.int32),
            pltpu.VMEM((K, D), jnp.float32),
            pltpu.SemaphoreType.DMA,
        ],
    )
    def prep(x_hbm, emb_hbm, h0_hbm, xb, eb, sem):
        c = lax.axis_index("c")
        s = lax.axis_index("s")
        wid = c * NS + s
        pltpu.sync_copy(x_hbm.at[wid], xb)
        for r, nr in ((0, K), (1, K), (2, RPT - 2 * K)):
            pltpu.async_copy(emb_hbm.at[xb.at[r]], eb, sem).wait()
            pltpu.sync_copy(eb.at[pl.ds(0, nr)],
                            h0_hbm.at[pl.ds(wid * RPT + r * K, nr)])

    return prep(x3, emb)


def _make_sc_agg():
    @functools.partial(
        pl.kernel,
        out_type=jax.ShapeDtypeStruct((NP, D), jnp.float32),
        mesh=_mesh(),
        scratch_types=[
            pltpu.VMEM((NB2, K), jnp.int32),
            pltpu.VMEM((NB2, K), jnp.int32),
            pltpu.VMEM((K,), jnp.int32),
            pltpu.VMEM((K, D), jnp.float32),
            pltpu.VMEM((K, D), jnp.float32),
            pltpu.VMEM_SHARED((ACC_R, D), jnp.float32),
            pltpu.SemaphoreType.DMA,
        ],
    )
    def agg(g_hbm, src_hbm, dst_hbm, out_hbm, srcb, dstb, dstloc, rows, ob,
            acc, sem):
        c = lax.axis_index("c")
        s = lax.axis_index("s")
        lob = c * SPLIT
        hib = SPLIT + c * (2 ** 24 - SPLIT)
        zv = jnp.zeros((16,), jnp.float32)

        def zrow(j, carry):
            for k in range(D // 16):
                ob[j, pl.ds(k * 16, 16)] = zv
            return carry

        lax.fori_loop(0, K, zrow, 0)
        for off, sz in ((0, K), (K, K), (2 * K, ZR - 2 * K)):
            pltpu.sync_copy(ob.at[pl.ds(0, sz)],
                            acc.at[pl.ds(s * ZR + off, sz)])
        plsc.subcore_barrier()

        pltpu.sync_copy(src_hbm.at[s], srcb)
        pltpu.sync_copy(dst_hbm.at[s], dstb)

        def body(b, carry):
            pltpu.async_copy(g_hbm.at[srcb.at[b]], rows, sem).wait()
            for k in range(K // 16):
                dv = dstb[b, pl.ds(k * 16, 16)]
                own = (dv >= lob) & (dv < hib)
                dstloc[pl.ds(k * 16, 16)] = jnp.where(own, dv - lob, SPLIT)
            pltpu.sync_copy(rows, acc.at[dstloc], add=True)
            return carry

        lax.fori_loop(0, NB2, body, 0)
        plsc.subcore_barrier()

        for off, sz in ((0, K), (K, K), (2 * K, WR - 2 * K)):
            pltpu.sync_copy(acc.at[pl.ds(s * WR + off, sz)],
                            ob.at[pl.ds(0, sz)])
            pltpu.sync_copy(ob.at[pl.ds(0, sz)],
                            out_hbm.at[pl.ds(c * SPLIT + s * WR + off, sz)])

    return agg


_sc_agg = _make_sc_agg()


def _sc_tgt(h4, tgt3):
    tw = T // NW

    @functools.partial(
        pl.kernel,
        out_type=jax.ShapeDtypeStruct((T, D), jnp.float32),
        mesh=_mesh(),
        scratch_types=[
            pltpu.VMEM((1, tw), jnp.int32),
            pltpu.VMEM((tw, D), jnp.float32),
            pltpu.SemaphoreType.DMA,
        ],
    )
    def tg(h_hbm, t_hbm, out_hbm, tb, rb, sem):
        c = lax.axis_index("c")
        s = lax.axis_index("s")
        wid = c * NS + s
        pltpu.sync_copy(t_hbm.at[wid], tb)
        pltpu.async_copy(h_hbm.at[tb.at[0]], rb, sem).wait()
        pltpu.sync_copy(rb, out_hbm.at[pl.ds(wid * tw, tw)])

    return tg(h4, tgt3)


def _tc_layer(a, ghat, degd, h0, flag, wn, cb, cg, cbeta):
    def body(a_ref, g_ref, dg_ref, h0_ref, f_ref, w_ref, cb_ref, cg_ref,
             cbe_ref, gn_ref, h_ref):
        dinv = lax.rsqrt(dg_ref[...] + 1.0)
        z = jnp.dot((a_ref[...] + g_ref[...]) * dinv, w_ref[...],
                    preferred_element_type=jnp.float32) + cb_ref[...]
        m = jnp.mean(z, axis=-1, keepdims=True)
        xc = z - m
        v = jnp.mean(xc * xc, axis=-1, keepdims=True)
        hh = jnp.maximum(xc * lax.rsqrt(v + EPS) * cg_ref[...] + cbe_ref[...],
                         0.0)
        h = jnp.where(f_ref[0, 0] > 0.0, h0_ref[...], hh)
        h_ref[...] = h
        gn_ref[...] = h * dinv

    return pl.pallas_call(
        body,
        grid=(GRID,),
        in_specs=[
            pl.BlockSpec((BR, D), lambda i: (i, 0)),
            pl.BlockSpec((BR, D), lambda i: (i, 0)),
            pl.BlockSpec((BR, 1), lambda i: (i, 0)),
            pl.BlockSpec((BR, D), lambda i: (i, 0)),
            pl.BlockSpec((1, 1), lambda i: (0, 0)),
            pl.BlockSpec((D, D), lambda i: (0, 0)),
            pl.BlockSpec((1, D), lambda i: (0, 0)),
            pl.BlockSpec((1, D), lambda i: (0, 0)),
            pl.BlockSpec((1, D), lambda i: (0, 0)),
        ],
        out_specs=(pl.BlockSpec((BR, D), lambda i: (i, 0)),
                   pl.BlockSpec((BR, D), lambda i: (i, 0))),
        out_shape=(jax.ShapeDtypeStruct((NP, D), jnp.float32),
                   jax.ShapeDtypeStruct((NP, D), jnp.float32)),
    )(a, ghat, degd, h0, flag, wn, cb, cg, cbeta)


def _tc_head(ht, lw, lb, lg, lbe, ow, obias):
    def body(h_ref, lw_ref, lb_ref, lg_ref, lbe_ref, ow_ref, ob_ref, o_ref):
        h = h_ref[...]
        for j in range(2):
            z = jnp.dot(h, lw_ref[j], preferred_element_type=jnp.float32) \
                + lb_ref[j]
            m = jnp.mean(z, axis=-1, keepdims=True)
            xc = z - m
            v = jnp.mean(xc * xc, axis=-1, keepdims=True)
            h = jnp.maximum(xc * lax.rsqrt(v + EPS) * lg_ref[j] + lbe_ref[j],
                            0.0)
        o_ref[...] = jnp.dot(h, ow_ref[...],
                             preferred_element_type=jnp.float32) + ob_ref[0, 0]

    return pl.pallas_call(
        body,
        grid=(1,),
        in_specs=[
            pl.BlockSpec((T, D), lambda i: (0, 0)),
            pl.BlockSpec((2, D, D), lambda i: (0, 0, 0)),
            pl.BlockSpec((2, 1, D), lambda i: (0, 0, 0)),
            pl.BlockSpec((2, 1, D), lambda i: (0, 0, 0)),
            pl.BlockSpec((2, 1, D), lambda i: (0, 0, 0)),
            pl.BlockSpec((D, D), lambda i: (0, 0)),
            pl.BlockSpec((1, 1), lambda i: (0, 0)),
        ],
        out_specs=pl.BlockSpec((T, D), lambda i: (0, 0)),
        out_shape=jax.ShapeDtypeStruct((T, D), jnp.float32),
    )(ht, lw, lb, lg, lbe, ow, obias)


def kernel(x, edge_index, edge_list, target_indices, emb, conv_W, conv_b,
           conv_g, conv_beta, lin_W, lin_b, lin_g, lin_beta, out_W, out_b):
    xi = x.ravel().astype(jnp.int32)
    xpad = jnp.concatenate([xi, jnp.zeros((NP - N,), jnp.int32)])
    x3 = jnp.pad(xpad.reshape(NW, RPT), ((0, 0), (0, XROWS * K - RPT))) \
        .reshape(NW, XROWS, K)
    edges = edge_list[0].astype(jnp.int32)
    src2 = jnp.concatenate(
        [edges[0], jnp.zeros((EP - E,), jnp.int32)]).reshape(NS, NB2, K)
    dst2 = jnp.concatenate(
        [edges[1], jnp.full((EP - E,), N, jnp.int32)]).reshape(NS, NB2, K)
    tgt3 = target_indices.ravel().astype(jnp.int32).reshape(NW, 1, T // NW)

    h0 = _sc_prep(x3, emb)

    idx = jnp.array([0, 0, 1, 2, 3], jnp.int32)
    w5 = conv_W[idx]
    cb5 = conv_b[idx].reshape(5, 1, D)
    cg5 = conv_g[idx].reshape(5, 1, D)
    cbe5 = conv_beta[idx].reshape(5, 1, D)
    flag5 = jnp.array([1.0, 0.0, 0.0, 0.0, 0.0],
                      jnp.float32).reshape(5, 1, 1)

    def step(carry, xs):
        ghat, h, degd = carry
        fl, wn, cb, cg, cbe = xs
        a = _sc_agg(ghat, src2, dst2)
        degd = jnp.where(fl[0, 0] > 0.0, a[:, 0:1], degd)
        gn, hn = _tc_layer(a, ghat, degd, h0, fl, wn, cb, cg, cbe)
        return (gn, hn, degd), None

    init = (jnp.ones((NP, D), jnp.float32), h0, jnp.zeros((NP, 1),
                                                          jnp.float32))
    (_, h4, _), _ = lax.scan(step, init, (flag5, w5, cb5, cg5, cbe5))
    ht = _sc_tgt(h4, tgt3)
    out = _tc_head(ht, lin_W, lin_b.reshape(2, 1, D), lin_g.reshape(2, 1, D),
                   lin_beta.reshape(2, 1, D),
                   jnp.pad(out_W, ((0, 0), (0, D - 1))), out_b.reshape(1, 1))
    return out[:, :1]


# speedup vs baseline: 5.0249x; 1.1195x over previous
import functools

import jax
import jax.numpy as jnp
from jax import lax
from jax.experimental import pallas as pl
from jax.experimental.pallas import tpu as pltpu
from jax.experimental.pallas import tpu_sc as plsc

N = 10000
NP = 10240
D = 128
E = 320000
T = 1024
EPS = 1e-5

NC = 2
NS = 16
NW = NC * NS
K = 128
NB2 = 158
EP = NS * NB2 * K
RPT = NP // NW
XROWS = 8

SPLIT = 5120
ACC_R = 5136
WR = SPLIT // NS
ZR = ACC_R // NS

BR = 512
GRID = NP // BR


def _mesh():
    return plsc.VectorSubcoreMesh(core_axis_name="c", subcore_axis_name="s")


def _sc_prep(x3, emb):
    @functools.partial(
        pl.kernel,
        out_type=jax.ShapeDtypeStruct((NP, D), jnp.float32),
        mesh=_mesh(),
        scratch_types=[
            pltpu.VMEM((XROWS, K), jnp.int32),
            pltpu.VMEM((K, D), jnp.float32),
            pltpu.SemaphoreType.DMA,
        ],
    )
    def prep(x_hbm, emb_hbm, h0_hbm, xb, eb, sem):
        c = lax.axis_index("c")
        s = lax.axis_index("s")
        wid = c * NS + s
        pltpu.sync_copy(x_hbm.at[wid], xb)
        for r, nr in ((0, K), (1, K), (2, RPT - 2 * K)):
            pltpu.async_copy(emb_hbm.at[xb.at[r]], eb, sem).wait()
            pltpu.sync_copy(eb.at[pl.ds(0, nr)],
                            h0_hbm.at[pl.ds(wid * RPT + r * K, nr)])

    return prep(x3, emb)


def _make_sc_agg():
    @functools.partial(
        pl.kernel,
        out_type=jax.ShapeDtypeStruct((NP, D), jnp.float32),
        mesh=_mesh(),
        scratch_types=[
            pltpu.VMEM((NB2, K), jnp.int32),
            pltpu.VMEM((NB2, K), jnp.int32),
            pltpu.VMEM((K,), jnp.int32),
            pltpu.VMEM((K, D), jnp.float32),
            pltpu.VMEM((K, D), jnp.float32),
            pltpu.VMEM_SHARED((ACC_R, D), jnp.float32),
            pltpu.SemaphoreType.DMA,
            pltpu.SemaphoreType.DMA,
        ],
    )
    def agg(g_hbm, src_hbm, dst_hbm, out_hbm, srcb, dstb, dstloc, rows0,
            ob, acc, sem0, sem1):
        rows1 = ob
        c = lax.axis_index("c")
        s = lax.axis_index("s")
        lob = c * SPLIT
        hib = SPLIT + c * (2 ** 24 - SPLIT)
        zv = jnp.zeros((16,), jnp.float32)

        def zrow(j, carry):
            for k in range(D // 16):
                ob[j, pl.ds(k * 16, 16)] = zv
            return carry

        lax.fori_loop(0, K, zrow, 0)
        for off, sz in ((0, K), (K, K), (2 * K, ZR - 2 * K)):
            pltpu.sync_copy(ob.at[pl.ds(0, sz)],
                            acc.at[pl.ds(s * ZR + off, sz)])
        plsc.subcore_barrier()

        pltpu.sync_copy(src_hbm.at[s], srcb)
        pltpu.sync_copy(dst_hbm.at[s], dstb)

        def start(b, rows, sem):
            pltpu.async_copy(g_hbm.at[srcb.at[b]], rows, sem)

        def scat(b, rows, sem):
            pltpu.make_async_copy(g_hbm.at[srcb.at[b]], rows, sem).wait()
            for k in range(K // 16):
                dv = dstb[b, pl.ds(k * 16, 16)]
                own = (dv >= lob) & (dv < hib)
                dstloc[pl.ds(k * 16, 16)] = jnp.where(own, dv - lob, SPLIT)
            pltpu.sync_copy(rows, acc.at[dstloc], add=True)

        start(0, rows0, sem0)

        def body(b2, carry):
            b = 2 * b2
            start(b + 1, rows1, sem1)
            scat(b, rows0, sem0)

            @pl.when(b2 < NB2 // 2 - 1)
            def _():
                start(b + 2, rows0, sem0)

            scat(b + 1, rows1, sem1)
            return carry

        lax.fori_loop(0, NB2 // 2, body, 0)
        plsc.subcore_barrier()

        for off, sz in ((0, K), (K, K), (2 * K, WR - 2 * K)):
            pltpu.sync_copy(acc.at[pl.ds(s * WR + off, sz)],
                            ob.at[pl.ds(0, sz)])
            pltpu.sync_copy(ob.at[pl.ds(0, sz)],
                            out_hbm.at[pl.ds(c * SPLIT + s * WR + off, sz)])

    return agg


_sc_agg = _make_sc_agg()


def _sc_tgt(h4, tgt3):
    tw = T // NW

    @functools.partial(
        pl.kernel,
        out_type=jax.ShapeDtypeStruct((T, D), jnp.float32),
        mesh=_mesh(),
        scratch_types=[
            pltpu.VMEM((1, tw), jnp.int32),
            pltpu.VMEM((tw, D), jnp.float32),
            pltpu.SemaphoreType.DMA,
        ],
    )
    def tg(h_hbm, t_hbm, out_hbm, tb, rb, sem):
        c = lax.axis_index("c")
        s = lax.axis_index("s")
        wid = c * NS + s
        pltpu.sync_copy(t_hbm.at[wid], tb)
        pltpu.async_copy(h_hbm.at[tb.at[0]], rb, sem).wait()
        pltpu.sync_copy(rb, out_hbm.at[pl.ds(wid * tw, tw)])

    return tg(h4, tgt3)


def _tc_layer(a, ghat, degd, h0, flag, wn, cb, cg, cbeta):
    def body(a_ref, g_ref, dg_ref, h0_ref, f_ref, w_ref, cb_ref, cg_ref,
             cbe_ref, gn_ref, h_ref):
        dinv = lax.rsqrt(dg_ref[...] + 1.0)
        z = jnp.dot((a_ref[...] + g_ref[...]) * dinv, w_ref[...],
                    preferred_element_type=jnp.float32) + cb_ref[...]
        m = jnp.mean(z, axis=-1, keepdims=True)
        xc = z - m
        v = jnp.mean(xc * xc, axis=-1, keepdims=True)
        hh = jnp.maximum(xc * lax.rsqrt(v + EPS) * cg_ref[...] + cbe_ref[...],
                         0.0)
        h = jnp.where(f_ref[0, 0] > 0.0, h0_ref[...], hh)
        h_ref[...] = h
        gn_ref[...] = h * dinv

    return pl.pallas_call(
        body,
        grid=(GRID,),
        in_specs=[
            pl.BlockSpec((BR, D), lambda i: (i, 0)),
            pl.BlockSpec((BR, D), lambda i: (i, 0)),
            pl.BlockSpec((BR, 1), lambda i: (i, 0)),
            pl.BlockSpec((BR, D), lambda i: (i, 0)),
            pl.BlockSpec((1, 1), lambda i: (0, 0)),
            pl.BlockSpec((D, D), lambda i: (0, 0)),
            pl.BlockSpec((1, D), lambda i: (0, 0)),
            pl.BlockSpec((1, D), lambda i: (0, 0)),
            pl.BlockSpec((1, D), lambda i: (0, 0)),
        ],
        out_specs=(pl.BlockSpec((BR, D), lambda i: (i, 0)),
                   pl.BlockSpec((BR, D), lambda i: (i, 0))),
        out_shape=(jax.ShapeDtypeStruct((NP, D), jnp.float32),
                   jax.ShapeDtypeStruct((NP, D), jnp.float32)),
    )(a, ghat, degd, h0, flag, wn, cb, cg, cbeta)


def _tc_head(ht, lw, lb, lg, lbe, ow, obias):
    def body(h_ref, lw_ref, lb_ref, lg_ref, lbe_ref, ow_ref, ob_ref, o_ref):
        h = h_ref[...]
        for j in range(2):
            z = jnp.dot(h, lw_ref[j], preferred_element_type=jnp.float32) \
                + lb_ref[j]
            m = jnp.mean(z, axis=-1, keepdims=True)
            xc = z - m
            v = jnp.mean(xc * xc, axis=-1, keepdims=True)
            h = jnp.maximum(xc * lax.rsqrt(v + EPS) * lg_ref[j] + lbe_ref[j],
                            0.0)
        o_ref[...] = jnp.dot(h, ow_ref[...],
                             preferred_element_type=jnp.float32) + ob_ref[0, 0]

    return pl.pallas_call(
        body,
        grid=(1,),
        in_specs=[
            pl.BlockSpec((T, D), lambda i: (0, 0)),
            pl.BlockSpec((2, D, D), lambda i: (0, 0, 0)),
            pl.BlockSpec((2, 1, D), lambda i: (0, 0, 0)),
            pl.BlockSpec((2, 1, D), lambda i: (0, 0, 0)),
            pl.BlockSpec((2, 1, D), lambda i: (0, 0, 0)),
            pl.BlockSpec((D, D), lambda i: (0, 0)),
            pl.BlockSpec((1, 1), lambda i: (0, 0)),
        ],
        out_specs=pl.BlockSpec((T, D), lambda i: (0, 0)),
        out_shape=jax.ShapeDtypeStruct((T, D), jnp.float32),
    )(ht, lw, lb, lg, lbe, ow, obias)


def kernel(x, edge_index, edge_list, target_indices, emb, conv_W, conv_b,
           conv_g, conv_beta, lin_W, lin_b, lin_g, lin_beta, out_W, out_b):
    xi = x.ravel().astype(jnp.int32)
    xpad = jnp.concatenate([xi, jnp.zeros((NP - N,), jnp.int32)])
    x3 = jnp.pad(xpad.reshape(NW, RPT), ((0, 0), (0, XROWS * K - RPT))) \
        .reshape(NW, XROWS, K)
    edges = edge_list[0].astype(jnp.int32)
    src2 = jnp.concatenate(
        [edges[0], jnp.zeros((EP - E,), jnp.int32)]).reshape(NS, NB2, K)
    dst2 = jnp.concatenate(
        [edges[1], jnp.full((EP - E,), N, jnp.int32)]).reshape(NS, NB2, K)
    tgt3 = target_indices.ravel().astype(jnp.int32).reshape(NW, 1, T // NW)

    h0 = _sc_prep(x3, emb)

    idx = jnp.array([0, 0, 1, 2, 3], jnp.int32)
    w5 = conv_W[idx]
    cb5 = conv_b[idx].reshape(5, 1, D)
    cg5 = conv_g[idx].reshape(5, 1, D)
    cbe5 = conv_beta[idx].reshape(5, 1, D)
    flag5 = jnp.array([1.0, 0.0, 0.0, 0.0, 0.0],
                      jnp.float32).reshape(5, 1, 1)

    def step(carry, xs):
        ghat, h, degd = carry
        fl, wn, cb, cg, cbe = xs
        a = _sc_agg(ghat, src2, dst2)
        degd = jnp.where(fl[0, 0] > 0.0, a[:, 0:1], degd)
        gn, hn = _tc_layer(a, ghat, degd, h0, fl, wn, cb, cg, cbe)
        return (gn, hn, degd), None

    init = (jnp.ones((NP, D), jnp.float32), h0, jnp.zeros((NP, 1),
                                                          jnp.float32))
    (_, h4, _), _ = lax.scan(step, init, (flag5, w5, cb5, cg5, cbe5))
    ht = _sc_tgt(h4, tgt3)
    out = _tc_head(ht, lin_W, lin_b.reshape(2, 1, D), lin_g.reshape(2, 1, D),
                   lin_beta.reshape(2, 1, D),
                   jnp.pad(out_W, ((0, 0), (0, D - 1))), out_b.reshape(1, 1))
    return out[:, :1]
